# Initial kernel scaffold; baseline (speedup 1.0000x reference)
import functools

import jax
import jax.numpy as jnp
from jax import lax
from jax.experimental import pallas as pl
from jax.experimental.pallas import tpu as pltpu
from jax.experimental.pallas import tpu_sc as plsc

_B_ROWS = 4096
_N_FEAT = 26
_D = 128

_info = plsc.get_sparse_core_info()
_NC, _NS = _info.num_cores, _info.num_subcores
_NW = _NC * _NS

_TOTAL = _B_ROWS * _N_FEAT
_PER_W = _TOTAL // _NW
_CHUNK = 416
_N_CHUNKS = _PER_W // _CHUNK


@functools.partial(
    pl.kernel,
    mesh=plsc.VectorSubcoreMesh(core_axis_name="c", subcore_axis_name="s"),
    out_type=jax.ShapeDtypeStruct((_TOTAL, _D), jnp.float32),
    scratch_types=[
        pltpu.VMEM((_PER_W,), jnp.int32),
        pltpu.VMEM((_CHUNK, _D), jnp.float32),
        pltpu.VMEM((_CHUNK, _D), jnp.float32),
        pltpu.SemaphoreType.DMA,
        pltpu.SemaphoreType.DMA,
    ],
)
def _gather_kernel(table_hbm, idx_hbm, out_hbm, idx_v, rows0, rows1, gsem, osem):
    wid = lax.axis_index("s") * _NC + lax.axis_index("c")
    base = wid * _PER_W
    pltpu.sync_copy(idx_hbm.at[pl.ds(base, _PER_W)], idx_v)

    bufs = (rows0, rows1)
    pltpu.async_copy(table_hbm.at[idx_v.at[pl.ds(0, _CHUNK)]], bufs[0], gsem)
    for c in range(_N_CHUNKS):
        buf = bufs[c % 2]
        nxt = bufs[(c + 1) % 2]
        if c + 1 < _N_CHUNKS:
            pltpu.async_copy(
                table_hbm.at[idx_v.at[pl.ds((c + 1) * _CHUNK, _CHUNK)]], nxt, gsem
            )
        pltpu.make_async_copy(
            table_hbm.at[idx_v.at[pl.ds(c * _CHUNK, _CHUNK)]], buf, gsem
        ).wait()
        out_slice = out_hbm.at[pl.ds(base + c * _CHUNK, _CHUNK)]
        pltpu.async_copy(buf, out_slice, osem)
        if c >= 1:
            prev = out_hbm.at[pl.ds(base + (c - 1) * _CHUNK, _CHUNK)]
            pltpu.make_async_copy(bufs[(c - 1) % 2], prev, osem).wait()
    last = _N_CHUNKS - 1
    pltpu.make_async_copy(
        bufs[last % 2], out_hbm.at[pl.ds(base + last * _CHUNK, _CHUNK)], osem
    ).wait()


def kernel(feat, emb_feat):
    flat = feat.reshape(-1).astype(jnp.int32)
    out = _gather_kernel(emb_feat, flat)
    return out.reshape(_B_ROWS, _N_FEAT, _D)


# speedup vs baseline: 1.2879x; 1.2879x over previous
import functools

import jax
import jax.numpy as jnp
from jax import lax
from jax.experimental import pallas as pl
from jax.experimental.pallas import tpu as pltpu
from jax.experimental.pallas import tpu_sc as plsc

_B_ROWS = 4096
_N_FEAT = 26
_D = 128

_info = plsc.get_sparse_core_info()
_NC, _NS = _info.num_cores, _info.num_subcores
_NW = _NC * _NS

_TOTAL = _B_ROWS * _N_FEAT
_PER_W = _TOTAL // _NW
_CHUNK = 416
_N_CHUNKS = _PER_W // _CHUNK


@functools.partial(
    pl.kernel,
    mesh=plsc.VectorSubcoreMesh(core_axis_name="c", subcore_axis_name="s"),
    out_type=jax.ShapeDtypeStruct((_TOTAL, _D), jnp.float32),
    scratch_types=[
        pltpu.VMEM((_PER_W,), jnp.int32),
        pltpu.VMEM((_CHUNK, _D), jnp.float32),
        pltpu.VMEM((_CHUNK, _D), jnp.float32),
        pltpu.SemaphoreType.DMA,
        pltpu.SemaphoreType.DMA,
        pltpu.SemaphoreType.DMA,
        pltpu.SemaphoreType.DMA,
    ],
)
def _gather_kernel(
    table_hbm, idx_hbm, out_hbm, idx_v, rows0, rows1, gsem0, gsem1, osem0, osem1
):
    wid = lax.axis_index("s") * _NC + lax.axis_index("c")
    base = wid * _PER_W
    pltpu.sync_copy(idx_hbm.at[pl.ds(base, _PER_W)], idx_v)

    bufs = (rows0, rows1)
    gsems = (gsem0, gsem1)
    osems = (osem0, osem1)
    pltpu.async_copy(table_hbm.at[idx_v.at[pl.ds(0, _CHUNK)]], bufs[0], gsems[0])
    for c in range(_N_CHUNKS):
        buf = bufs[c % 2]
        if c + 1 < _N_CHUNKS:
            nb = (c + 1) % 2
            if c >= 1:
                pltpu.make_async_copy(
                    bufs[nb],
                    out_hbm.at[pl.ds(base + (c - 1) * _CHUNK, _CHUNK)],
                    osems[nb],
                ).wait()
            pltpu.async_copy(
                table_hbm.at[idx_v.at[pl.ds((c + 1) * _CHUNK, _CHUNK)]],
                bufs[nb],
                gsems[nb],
            )
        pltpu.make_async_copy(
            table_hbm.at[idx_v.at[pl.ds(c * _CHUNK, _CHUNK)]], buf, gsems[c % 2]
        ).wait()
        pltpu.async_copy(buf, out_hbm.at[pl.ds(base + c * _CHUNK, _CHUNK)], osems[c % 2])
    for c in (_N_CHUNKS - 2, _N_CHUNKS - 1):
        pltpu.make_async_copy(
            bufs[c % 2], out_hbm.at[pl.ds(base + c * _CHUNK, _CHUNK)], osems[c % 2]
        ).wait()


def kernel(feat, emb_feat):
    flat = feat.reshape(-1).astype(jnp.int32)
    out = _gather_kernel(emb_feat, flat)
    return out.reshape(_B_ROWS, _N_FEAT, _D)
